# baseline (speedup 1.0000x reference)
import functools

import jax
import jax.numpy as jnp
from jax import lax
from jax.experimental import pallas as pl
from jax.experimental.pallas import tpu as pltpu
from jax.experimental.pallas import tpu_sc as plsc

M = 4
IN_DIM = 512
DIM = 256
K = 1024
B = 4096
EPS = 1e-5
BT = 512
NB = B // BT

_SC_INFO = plsc.get_sparse_core_info()
_NC = _SC_INFO.num_cores
_NS = _SC_INFO.num_subcores
_NW = _NC * _NS
_ROWS_PER_W = (M * B) // _NW
_CHUNK = 128
_NCHUNK = _ROWS_PER_W // _CHUNK


def _bn(h, g, b):
    return (h / jnp.sqrt(1.0 + EPS)) * g + b


def _enc_body(x_ref, eW1, eb1, eg1, ebe1, eW2, eb2, eg2, ebe2, eW3, eb3, cbs,
              res_ref, gidx_ref):
    x = x_ref[...]
    for m in range(M):
        h = lax.dot_general(x, eW1[m], (((1,), (1,)), ((), ())),
                            preferred_element_type=jnp.float32) + eb1[m:m + 1, :]
        h = jnp.maximum(_bn(h, eg1[m:m + 1, :], ebe1[m:m + 1, :]), 0.0)
        h = lax.dot_general(h, eW2[m], (((1,), (1,)), ((), ())),
                            preferred_element_type=jnp.float32) + eb2[m:m + 1, :]
        h = jnp.maximum(_bn(h, eg2[m:m + 1, :], ebe2[m:m + 1, :]), 0.0)
        ze = lax.dot_general(h, eW3[m], (((1,), (1,)), ((), ())),
                             preferred_element_type=jnp.float32) + eb3[m:m + 1, :]
        res_ref[m] = ze
        emb = cbs[m]
        a = jnp.sum(ze * ze, axis=1)[:, None]
        bb = jnp.sum(emb * emb, axis=1)[None, :]
        c = lax.dot_general(ze, emb, (((1,), (1,)), ((), ())),
                            preferred_element_type=jnp.float32)
        dist = (a + bb) - 2.0 * c
        minv = jnp.min(dist, axis=1, keepdims=True)
        iota = lax.broadcasted_iota(jnp.int32, (BT, K), 1)
        nn = jnp.min(jnp.where(dist == minv, iota, K), axis=1)
        gidx_ref[m] = nn


def _dec_body(ce_ref, dW1, db1, dg1, dbe1, dW2, db2, dg2, dbe2, dW3, db3,
              xhat_ref):
    zq = ce_ref[0] + ce_ref[1] + ce_ref[2] + ce_ref[3]
    d = lax.dot_general(zq, dW1[...], (((1,), (1,)), ((), ())),
                        preferred_element_type=jnp.float32) + db1[...]
    d = jnp.maximum(_bn(d, dg1[...], dbe1[...]), 0.0)
    d = lax.dot_general(d, dW2[...], (((1,), (1,)), ((), ())),
                        preferred_element_type=jnp.float32) + db2[...]
    d = jnp.maximum(_bn(d, dg2[...], dbe2[...]), 0.0)
    xhat_ref[...] = lax.dot_general(d, dW3[...], (((1,), (1,)), ((), ())),
                                    preferred_element_type=jnp.float32) + db3[...]


_sc_mesh = plsc.VectorSubcoreMesh(core_axis_name="c", subcore_axis_name="s")


_ROWS_PER_TEC = 2048
_OUT_CHUNK = 128
_CQ = 64
_CQ16 = 2 * _CQ


@functools.partial(
    pl.kernel,
    mesh=_sc_mesh,
    out_type=jax.ShapeDtypeStruct((M * B, 2 * DIM), jnp.uint16),
    scratch_types=[
        pltpu.VMEM((K, _CQ16), jnp.uint16),
        pltpu.VMEM((_ROWS_PER_TEC,), jnp.int32),
        pltpu.VMEM((_OUT_CHUNK, _CQ16), jnp.uint16),
        pltpu.VMEM((_OUT_CHUNK, _CQ16), jnp.uint16),
        pltpu.SemaphoreType.DMA,
        pltpu.SemaphoreType.DMA,
        pltpu.SemaphoreType.DMA,
    ],
)
def _sc_gather(table_hbm, idx_hbm, out_hbm, tbl16, idx_v, ob0, ob1,
               s0, s1, si):
    wid = lax.axis_index("s") * _NC + lax.axis_index("c")
    m = wid // 8
    r = wid % 8
    h = r // 4
    q = r % 4
    rowbase = m * B + h * _ROWS_PER_TEC
    idxcp = pltpu.async_copy(
        idx_hbm.at[pl.ds(rowbase, _ROWS_PER_TEC)], idx_v, si)
    pltpu.sync_copy(
        table_hbm.at[pl.ds(m * K, K), pl.ds(q * _CQ16, _CQ16)], tbl16)
    tblf = tbl16.bitcast(jnp.float32)
    obf = (ob0.bitcast(jnp.float32), ob1.bitcast(jnp.float32))
    obufs = (ob0, ob1)
    ssems = (s0, s1)
    idxcp.wait()
    cps = [None, None]
    for chunk in range(_ROWS_PER_TEC // _OUT_CHUNK):
        bb = chunk % 2
        if cps[bb] is not None:
            cps[bb].wait()

        def body(g, _, _c=chunk, _o=obf[bb]):
            rvec = idx_v[pl.ds(_c * _OUT_CHUNK + g * 16, 16)]
            for l in range(16):
                row = rvec[l]
                for cb in range(_CQ // 16):
                    _o[g * 16 + l, pl.ds(cb * 16, 16)] = (
                        tblf[row, pl.ds(cb * 16, 16)])
            return 0

        lax.fori_loop(0, _OUT_CHUNK // 16, body, 0)
        cps[bb] = pltpu.async_copy(
            obufs[bb],
            out_hbm.at[pl.ds(rowbase + chunk * _OUT_CHUNK, _OUT_CHUNK),
                       pl.ds(q * _CQ16, _CQ16)], ssems[bb])
    for bb in range(2):
        if cps[bb] is not None:
            cps[bb].wait()


def kernel(x, enc_W1, enc_b1, enc_g1, enc_be1, enc_W2, enc_b2, enc_g2, enc_be2,
           enc_W3, enc_b3, codebooks, dec_W1, dec_b1, dec_g1, dec_be1,
           dec_W2, dec_b2, dec_g2, dec_be2, dec_W3, dec_b3):
    full = lambda shape: pl.BlockSpec(shape, lambda i: (0,) * len(shape))
    res, gidx = pl.pallas_call(
        _enc_body,
        grid=(NB,),
        in_specs=[
            pl.BlockSpec((BT, IN_DIM), lambda i: (i, 0)),
            full((M, 128, IN_DIM)), full((M, 128)), full((M, 128)), full((M, 128)),
            full((M, 256, 128)), full((M, 256)), full((M, 256)), full((M, 256)),
            full((M, DIM, 256)), full((M, DIM)),
            full((M, K, DIM)),
        ],
        out_specs=[
            pl.BlockSpec((M, BT, DIM), lambda i: (0, i, 0)),
            pl.BlockSpec((M, BT), lambda i: (0, i)),
        ],
        out_shape=[
            jax.ShapeDtypeStruct((M, B, DIM), jnp.float32),
            jax.ShapeDtypeStruct((M, B), jnp.int32),
        ],
    )(x, enc_W1, enc_b1, enc_g1, enc_be1, enc_W2, enc_b2, enc_g2, enc_be2,
      enc_W3, enc_b3, codebooks)

    table_u16 = lax.bitcast_convert_type(codebooks, jnp.uint16)
    table_u16 = table_u16.reshape(M * K, 2 * DIM)
    ce16 = _sc_gather(table_u16, gidx.reshape(M * B))
    ce = lax.bitcast_convert_type(ce16.reshape(M, B, DIM, 2), jnp.float32)

    x_hat = pl.pallas_call(
        _dec_body,
        grid=(NB,),
        in_specs=[
            pl.BlockSpec((M, BT, DIM), lambda i: (0, i, 0)),
            full((256, DIM)), full((256,)), full((256,)), full((256,)),
            full((128, 256)), full((128,)), full((128,)), full((128,)),
            full((IN_DIM, 128)), full((IN_DIM,)),
        ],
        out_specs=pl.BlockSpec((BT, IN_DIM), lambda i: (i, 0)),
        out_shape=jax.ShapeDtypeStruct((B, IN_DIM), jnp.float32),
    )(ce, dec_W1, dec_b1, dec_g1, dec_be1, dec_W2, dec_b2, dec_g2, dec_be2,
      dec_W3, dec_b3)
    return (x_hat, res, ce)


# speedup vs baseline: 1.2217x; 1.2217x over previous
import functools

import jax
import jax.numpy as jnp
from jax import lax
from jax.experimental import pallas as pl
from jax.experimental.pallas import tpu as pltpu
from jax.experimental.pallas import tpu_sc as plsc

M = 4
MB2 = 2
IN_DIM = 512
DIM = 256
K = 1024
B = 4096
EPS = 1e-5
BT = 512
NB = B // BT

_SC_INFO = plsc.get_sparse_core_info()
_NC = _SC_INFO.num_cores
_NS = _SC_INFO.num_subcores
_NW = _NC * _NS
_ROWS_PER_W = (MB2 * B) // _NW
_CHUNK = 128
_NCHUNK = _ROWS_PER_W // _CHUNK


def _bn(h, g, b):
    return (h / jnp.sqrt(1.0 + EPS)) * g + b


def _enc_body(x_ref, eW1, eb1, eg1, ebe1, eW2, eb2, eg2, ebe2, eW3, eb3, cbs,
              res_ref, gidx_ref):
    x = x_ref[...]
    for m in range(MB2):
        h = lax.dot_general(x, eW1[m], (((1,), (1,)), ((), ())),
                            preferred_element_type=jnp.float32) + eb1[m:m + 1, :]
        h = jnp.maximum(_bn(h, eg1[m:m + 1, :], ebe1[m:m + 1, :]), 0.0)
        h = lax.dot_general(h, eW2[m], (((1,), (1,)), ((), ())),
                            preferred_element_type=jnp.float32) + eb2[m:m + 1, :]
        h = jnp.maximum(_bn(h, eg2[m:m + 1, :], ebe2[m:m + 1, :]), 0.0)
        ze = lax.dot_general(h, eW3[m], (((1,), (1,)), ((), ())),
                             preferred_element_type=jnp.float32) + eb3[m:m + 1, :]
        res_ref[m] = ze
        emb = cbs[m]
        a = jnp.sum(ze * ze, axis=1)[:, None]
        bb = jnp.sum(emb * emb, axis=1)[None, :]
        c = lax.dot_general(ze, emb, (((1,), (1,)), ((), ())),
                            preferred_element_type=jnp.float32)
        dist = (a + bb) - 2.0 * c
        minv = jnp.min(dist, axis=1, keepdims=True)
        iota = lax.broadcasted_iota(jnp.int32, (BT, K), 1)
        nn = jnp.min(jnp.where(dist == minv, iota, K), axis=1)
        gidx_ref[m] = nn + m * K


def _dec_body(ce0_ref, ce1_ref, dW1, db1, dg1, dbe1, dW2, db2, dg2, dbe2,
              dW3, db3, xhat_ref):
    zq = ((ce0_ref[0] + ce0_ref[1]) + ce1_ref[0]) + ce1_ref[1]
    d = lax.dot_general(zq, dW1[...], (((1,), (1,)), ((), ())),
                        preferred_element_type=jnp.float32) + db1[...]
    d = jnp.maximum(_bn(d, dg1[...], dbe1[...]), 0.0)
    d = lax.dot_general(d, dW2[...], (((1,), (1,)), ((), ())),
                        preferred_element_type=jnp.float32) + db2[...]
    d = jnp.maximum(_bn(d, dg2[...], dbe2[...]), 0.0)
    xhat_ref[...] = lax.dot_general(d, dW3[...], (((1,), (1,)), ((), ())),
                                    preferred_element_type=jnp.float32) + db3[...]


_sc_mesh = plsc.VectorSubcoreMesh(core_axis_name="c", subcore_axis_name="s")


@functools.partial(
    pl.kernel,
    mesh=_sc_mesh,
    out_type=jax.ShapeDtypeStruct((MB2 * B, DIM), jnp.float32),
    scratch_types=[
        pltpu.VMEM((_ROWS_PER_W,), jnp.int32),
        pltpu.VMEM((_CHUNK, DIM), jnp.float32),
        pltpu.VMEM((_CHUNK, DIM), jnp.float32),
        pltpu.SemaphoreType.DMA,
        pltpu.SemaphoreType.DMA,
    ],
)
def _sc_gather(table_hbm, idx_hbm, out_hbm, idx_v, r0, r1, g0, g1):
    bufs = (r0, r1)
    gsems = (g0, g1)
    wid = lax.axis_index("s") * _NC + lax.axis_index("c")
    base = wid * _ROWS_PER_W
    pltpu.sync_copy(idx_hbm.at[pl.ds(base, _ROWS_PER_W)], idx_v)
    gcp = [None, None]
    for j in range(min(2, _NCHUNK)):
        gcp[j] = pltpu.async_copy(
            table_hbm.at[idx_v.at[pl.ds(j * _CHUNK, _CHUNK)]], bufs[j], gsems[j])
    for j in range(_NCHUNK):
        b = j % 2
        gcp[b].wait()
        pltpu.sync_copy(bufs[b], out_hbm.at[pl.ds(base + j * _CHUNK, _CHUNK)])


def _enc_call(xs, eW1, eb1, eg1, ebe1, eW2, eb2, eg2, ebe2, eW3, eb3, cbs):
    full = lambda shape: pl.BlockSpec(shape, lambda i: (0,) * len(shape))
    return pl.pallas_call(
        _enc_body,
        grid=(NB,),
        in_specs=[
            pl.BlockSpec((BT, IN_DIM), lambda i: (i, 0)),
            full((MB2, 128, IN_DIM)), full((MB2, 128)), full((MB2, 128)),
            full((MB2, 128)),
            full((MB2, 256, 128)), full((MB2, 256)), full((MB2, 256)),
            full((MB2, 256)),
            full((MB2, DIM, 256)), full((MB2, DIM)),
            full((MB2, K, DIM)),
        ],
        out_specs=[
            pl.BlockSpec((MB2, BT, DIM), lambda i: (0, i, 0)),
            pl.BlockSpec((MB2, BT), lambda i: (0, i)),
        ],
        out_shape=[
            jax.ShapeDtypeStruct((MB2, B, DIM), jnp.float32),
            jax.ShapeDtypeStruct((MB2, B), jnp.int32),
        ],
    )(xs, eW1, eb1, eg1, ebe1, eW2, eb2, eg2, ebe2, eW3, eb3, cbs)


def kernel(x, enc_W1, enc_b1, enc_g1, enc_be1, enc_W2, enc_b2, enc_g2, enc_be2,
           enc_W3, enc_b3, codebooks, dec_W1, dec_b1, dec_g1, dec_be1,
           dec_W2, dec_b2, dec_g2, dec_be2, dec_W3, dec_b3):
    full = lambda shape: pl.BlockSpec(shape, lambda i: (0,) * len(shape))
    res_list, gidx_list, ce_list = [], [], []
    for p in range(2):
        sl = slice(p * MB2, (p + 1) * MB2)
        res_p, gidx_p = _enc_call(
            x, enc_W1[sl], enc_b1[sl], enc_g1[sl], enc_be1[sl],
            enc_W2[sl], enc_b2[sl], enc_g2[sl], enc_be2[sl],
            enc_W3[sl], enc_b3[sl], codebooks[sl])
        res_list.append(res_p)
        gidx_list.append(gidx_p)
    for p in range(2):
        sl = slice(p * MB2, (p + 1) * MB2)
        table_p = codebooks[sl].reshape(MB2 * K, DIM)
        ce_p = _sc_gather(table_p, gidx_list[p].reshape(MB2 * B))
        ce_list.append(ce_p.reshape(MB2, B, DIM))

    res = jnp.concatenate(res_list, axis=0)
    ce = jnp.concatenate(ce_list, axis=0)

    x_hat = pl.pallas_call(
        _dec_body,
        grid=(NB,),
        in_specs=[
            pl.BlockSpec((MB2, BT, DIM), lambda i: (0, i, 0)),
            pl.BlockSpec((MB2, BT, DIM), lambda i: (0, i, 0)),
            full((256, DIM)), full((256,)), full((256,)), full((256,)),
            full((128, 256)), full((128,)), full((128,)), full((128,)),
            full((IN_DIM, 128)), full((IN_DIM,)),
        ],
        out_specs=pl.BlockSpec((BT, IN_DIM), lambda i: (i, 0)),
        out_shape=jax.ShapeDtypeStruct((B, IN_DIM), jnp.float32),
    )(ce_list[0], ce_list[1], dec_W1, dec_b1, dec_g1, dec_be1,
      dec_W2, dec_b2, dec_g2, dec_be2, dec_W3, dec_b3)
    return (x_hat, res, ce)


# speedup vs baseline: 1.5982x; 1.3082x over previous
import functools

import jax
import jax.numpy as jnp
from jax import lax
from jax.experimental import pallas as pl
from jax.experimental.pallas import tpu as pltpu
from jax.experimental.pallas import tpu_sc as plsc

M = 4
IN_DIM = 512
DIM = 256
K = 1024
B = 4096
EPS = 1e-5
BT = 512
NB = B // BT

_SC_INFO = plsc.get_sparse_core_info()
_NC = _SC_INFO.num_cores
_NS = _SC_INFO.num_subcores
_NW = _NC * _NS
_ROWS_PER_W = (M * B) // _NW
_CHUNK = 128
_NCHUNK = _ROWS_PER_W // _CHUNK


def _bn(h, g, b):
    return (h / jnp.sqrt(1.0 + EPS)) * g + b


def _enc_body(x_ref, eW1, eb1, eg1, ebe1, eW2, eb2, eg2, ebe2, eW3, eb3, cbs,
              res_ref, gidx_ref):
    x = x_ref[...]
    for m in range(M):
        h = lax.dot_general(x, eW1[m], (((1,), (1,)), ((), ())),
                            preferred_element_type=jnp.float32) + eb1[m:m + 1, :]
        h = jnp.maximum(_bn(h, eg1[m:m + 1, :], ebe1[m:m + 1, :]), 0.0)
        h = lax.dot_general(h, eW2[m], (((1,), (1,)), ((), ())),
                            preferred_element_type=jnp.float32) + eb2[m:m + 1, :]
        h = jnp.maximum(_bn(h, eg2[m:m + 1, :], ebe2[m:m + 1, :]), 0.0)
        ze = lax.dot_general(h, eW3[m], (((1,), (1,)), ((), ())),
                             preferred_element_type=jnp.float32) + eb3[m:m + 1, :]
        res_ref[m] = ze
        emb = cbs[m]
        a = jnp.sum(ze * ze, axis=1)[:, None]
        bb = jnp.sum(emb * emb, axis=1)[None, :]
        c = lax.dot_general(ze, emb, (((1,), (1,)), ((), ())),
                            preferred_element_type=jnp.float32)
        dist = (a + bb) - 2.0 * c
        minv = jnp.min(dist, axis=1, keepdims=True)
        iota = lax.broadcasted_iota(jnp.int32, (BT, K), 1)
        nn = jnp.min(jnp.where(dist == minv, iota, K), axis=1)
        gidx_ref[m] = nn + m * K


def _dec_body(ce_ref, dW1, db1, dg1, dbe1, dW2, db2, dg2, dbe2, dW3, db3,
              xhat_ref):
    zq = ce_ref[0] + ce_ref[1] + ce_ref[2] + ce_ref[3]
    d = lax.dot_general(zq, dW1[...], (((1,), (1,)), ((), ())),
                        preferred_element_type=jnp.float32) + db1[...]
    d = jnp.maximum(_bn(d, dg1[...], dbe1[...]), 0.0)
    d = lax.dot_general(d, dW2[...], (((1,), (1,)), ((), ())),
                        preferred_element_type=jnp.float32) + db2[...]
    d = jnp.maximum(_bn(d, dg2[...], dbe2[...]), 0.0)
    xhat_ref[...] = lax.dot_general(d, dW3[...], (((1,), (1,)), ((), ())),
                                    preferred_element_type=jnp.float32) + db3[...]


_sc_mesh = plsc.VectorSubcoreMesh(core_axis_name="c", subcore_axis_name="s")


@functools.partial(
    pl.kernel,
    mesh=_sc_mesh,
    out_type=jax.ShapeDtypeStruct((M * B, DIM), jnp.float32),
    scratch_types=[
        pltpu.VMEM((_CHUNK,), jnp.int32),
        pltpu.VMEM((_CHUNK, DIM), jnp.float32),
        pltpu.SemaphoreType.DMA,
    ],
)
def _sc_gather(table_hbm, idx_hbm, out_hbm, idx_v, rows_v, sem):
    wid = lax.axis_index("s") * _NC + lax.axis_index("c")
    base = wid * _ROWS_PER_W
    for j in range(_NCHUNK):
        off = base + j * _CHUNK
        pltpu.sync_copy(idx_hbm.at[pl.ds(off, _CHUNK)], idx_v)
        pltpu.async_copy(table_hbm.at[idx_v], rows_v, sem).wait()
        pltpu.sync_copy(rows_v, out_hbm.at[pl.ds(off, _CHUNK)])


def kernel(x, enc_W1, enc_b1, enc_g1, enc_be1, enc_W2, enc_b2, enc_g2, enc_be2,
           enc_W3, enc_b3, codebooks, dec_W1, dec_b1, dec_g1, dec_be1,
           dec_W2, dec_b2, dec_g2, dec_be2, dec_W3, dec_b3):
    full = lambda shape: pl.BlockSpec(shape, lambda i: (0,) * len(shape))
    res, gidx = pl.pallas_call(
        _enc_body,
        grid=(NB,),
        in_specs=[
            pl.BlockSpec((BT, IN_DIM), lambda i: (i, 0)),
            full((M, 128, IN_DIM)), full((M, 128)), full((M, 128)), full((M, 128)),
            full((M, 256, 128)), full((M, 256)), full((M, 256)), full((M, 256)),
            full((M, DIM, 256)), full((M, DIM)),
            full((M, K, DIM)),
        ],
        out_specs=[
            pl.BlockSpec((M, BT, DIM), lambda i: (0, i, 0)),
            pl.BlockSpec((M, BT), lambda i: (0, i)),
        ],
        out_shape=[
            jax.ShapeDtypeStruct((M, B, DIM), jnp.float32),
            jax.ShapeDtypeStruct((M, B), jnp.int32),
        ],
    )(x, enc_W1, enc_b1, enc_g1, enc_be1, enc_W2, enc_b2, enc_g2, enc_be2,
      enc_W3, enc_b3, codebooks)

    table = codebooks.reshape(M * K, DIM)
    ce_flat = _sc_gather(table, gidx.reshape(M * B))
    ce = ce_flat.reshape(M, B, DIM)

    x_hat = pl.pallas_call(
        _dec_body,
        grid=(NB,),
        in_specs=[
            pl.BlockSpec((M, BT, DIM), lambda i: (0, i, 0)),
            full((256, DIM)), full((256,)), full((256,)), full((256,)),
            full((128, 256)), full((128,)), full((128,)), full((128,)),
            full((IN_DIM, 128)), full((IN_DIM,)),
        ],
        out_specs=pl.BlockSpec((BT, IN_DIM), lambda i: (i, 0)),
        out_shape=jax.ShapeDtypeStruct((B, IN_DIM), jnp.float32),
    )(ce, dec_W1, dec_b1, dec_g1, dec_be1, dec_W2, dec_b2, dec_g2, dec_be2,
      dec_W3, dec_b3)
    return (x_hat, res, ce)
